# baseline (speedup 1.0000x reference)
import jax
import jax.numpy as jnp
from jax import lax
from jax.experimental import pallas as pl
from jax.experimental.pallas import tpu as pltpu
from jax.experimental.pallas import tpu_sc as plsc

N = 10000
NPAD = 10240
E = 320000
D = 128

NC = 2
NS = 16
NW = NC * NS

CHUNK = 64
P = 160
NCHUNKS = NW * P
EPAD = NCHUNKS * CHUNK
NB = 1
GROUP = NB * CHUNK
NG = P // NB
ROWS_PER_TILE = NPAD // NS
DEGW = 16
DEGWIN = 16

_mesh = plsc.VectorSubcoreMesh(core_axis_name="c", subcore_axis_name="s")

_sc_params = pltpu.CompilerParams(use_tc_tiling_on_sc=False)


def _deg_body(col2d_hbm, zeros8_hbm, ones8_hbm, degp_hbm,
              deg_sh, cidx_v, ones_v, dsem):
    c = lax.axis_index("c")
    s = lax.axis_index("s")
    wid = s * NC + c
    pltpu.sync_copy(zeros8_hbm, deg_sh.at[pl.ds(s * ROWS_PER_TILE, ROWS_PER_TILE)])
    pltpu.sync_copy(ones8_hbm, ones_v)
    pltpu.sync_copy(col2d_hbm.at[pl.ds(wid * P, P)], cidx_v)
    plsc.subcore_barrier()

    def body(j, _):
        @pl.when(j < P)
        def _():
            pltpu.async_copy(ones_v, deg_sh.at[cidx_v.at[j]], dsem, add=True)

        @pl.when(j >= DEGWIN)
        def _():
            pltpu.make_async_copy(ones8_hbm, ones_v, dsem).wait()

        return None

    lax.fori_loop(0, P + DEGWIN, body, None)
    plsc.subcore_barrier()
    pltpu.sync_copy(
        deg_sh.at[pl.ds(s * ROWS_PER_TILE, ROWS_PER_TILE)],
        degp_hbm.at[c, pl.ds(s * ROWS_PER_TILE, ROWS_PER_TILE)],
    )


_deg_call = pl.kernel(
    _deg_body,
    out_type=jax.ShapeDtypeStruct((NC, NPAD, DEGW), jnp.float32),
    mesh=_mesh,
    scratch_types=[
        pltpu.VMEM_SHARED((NPAD, DEGW), jnp.float32),
        pltpu.VMEM((P, CHUNK), jnp.int32),
        pltpu.VMEM((CHUNK, DEGW), jnp.float32),
        pltpu.SemaphoreType.DMA,
    ],
    compiler_params=_sc_params,
)


def _agg_body(row2d_hbm, col2d_hbm, xs_hbm, zerosd_hbm, aggp_hbm,
              acc_sh, ridx_v, cidx_v, rows_v, gsem0, gsem1, ssem0, ssem1):
    c = lax.axis_index("c")
    s = lax.axis_index("s")
    wid = s * NC + c
    pltpu.sync_copy(row2d_hbm.at[pl.ds(wid * P, P)], ridx_v)
    pltpu.sync_copy(col2d_hbm.at[pl.ds(wid * P, P)], cidx_v)
    pltpu.sync_copy(zerosd_hbm, acc_sh.at[pl.ds(s * ROWS_PER_TILE, ROWS_PER_TILE)])
    plsc.subcore_barrier()

    def gather_group(g, slot, gsem):
        for k in range(NB):
            pltpu.async_copy(
                xs_hbm.at[ridx_v.at[g * NB + k]],
                rows_v.at[slot, pl.ds(k * CHUNK, CHUNK)],
                gsem,
            )

    def scatter_group(g, slot, ssem):
        for k in range(NB):
            pltpu.async_copy(
                rows_v.at[slot, pl.ds(k * CHUNK, CHUNK)],
                acc_sh.at[cidx_v.at[g * NB + k]],
                ssem,
                add=True,
            )

    def drain(slot, sem):
        pltpu.make_async_copy(xs_hbm.at[pl.ds(0, GROUP)], rows_v.at[slot], sem).wait()

    gather_group(0, 0, gsem0)

    def real_body(t, _):
        g0 = 2 * t
        g1 = g0 + 1

        @pl.when(t >= 1)
        def _():
            drain(1, ssem1)

        gather_group(g1, 1, gsem1)
        drain(0, gsem0)
        scatter_group(g0, 0, ssem0)

        drain(0, ssem0)

        @pl.when(t < NG // 2 - 1)
        def _():
            gather_group(g1 + 1, 0, gsem0)

        drain(1, gsem1)
        scatter_group(g1, 1, ssem1)
        return None

    lax.fori_loop(0, NG // 2, real_body, None)
    drain(1, ssem1)
    plsc.subcore_barrier()
    pltpu.sync_copy(
        acc_sh.at[pl.ds(s * ROWS_PER_TILE, ROWS_PER_TILE)],
        aggp_hbm.at[c, pl.ds(s * ROWS_PER_TILE, ROWS_PER_TILE)],
    )


_agg_call = pl.kernel(
    _agg_body,
    out_type=jax.ShapeDtypeStruct((NC, NPAD, D), jnp.float32),
    mesh=_mesh,
    scratch_types=[
        pltpu.VMEM_SHARED((NPAD, D), jnp.float32),
        pltpu.VMEM((P, CHUNK), jnp.int32),
        pltpu.VMEM((P, CHUNK), jnp.int32),
        pltpu.VMEM((2, GROUP, D), jnp.float32),
        pltpu.SemaphoreType.DMA,
        pltpu.SemaphoreType.DMA,
        pltpu.SemaphoreType.DMA,
        pltpu.SemaphoreType.DMA,
    ],
    compiler_params=_sc_params,
)

_BN = 1024


def _prescale_body(degp_ref, x_ref, xs_ref):
    deg = degp_ref[0, :, 0:1] + degp_ref[1, :, 0:1]
    dis = jnp.where(deg > 0.0, lax.rsqrt(deg), 0.0)
    xs_ref[...] = x_ref[...] * dis


def _prescale(degp, x):
    return pl.pallas_call(
        _prescale_body,
        grid=(NPAD // _BN,),
        in_specs=[
            pl.BlockSpec((NC, _BN, DEGW), lambda i: (0, i, 0)),
            pl.BlockSpec((_BN, D), lambda i: (i, 0)),
        ],
        out_specs=pl.BlockSpec((_BN, D), lambda i: (i, 0)),
        out_shape=jax.ShapeDtypeStruct((NPAD, D), jnp.float32),
    )(degp, x)


def _final_body(degp_ref, aggp_ref, w_ref, b_ref, o_ref):
    deg = degp_ref[0, :, 0:1] + degp_ref[1, :, 0:1]
    dis = jnp.where(deg > 0.0, lax.rsqrt(deg), 0.0)
    acc = (aggp_ref[0] + aggp_ref[1]) * dis
    o_ref[...] = lax.dot_general(
        acc, w_ref[...], (((1,), (1,)), ((), ())),
        preferred_element_type=jnp.float32,
    ) + b_ref[...]


def _final(degp, aggp, W_w, W_b2):
    return pl.pallas_call(
        _final_body,
        grid=(NPAD // _BN,),
        in_specs=[
            pl.BlockSpec((NC, _BN, DEGW), lambda i: (0, i, 0)),
            pl.BlockSpec((NC, _BN, D), lambda i: (0, i, 0)),
            pl.BlockSpec((D, D), lambda i: (0, 0)),
            pl.BlockSpec((1, D), lambda i: (0, 0)),
        ],
        out_specs=pl.BlockSpec((_BN, D), lambda i: (i, 0)),
        out_shape=jax.ShapeDtypeStruct((NPAD, D), jnp.float32),
    )(degp, aggp, W_w, W_b2)


@jax.jit
def kernel(x, edge_index, x0, W_w, W_b):
    del x0
    pad = jnp.full((EPAD - E,), N, jnp.int32)
    row2d = jnp.concatenate([edge_index[0], pad]).reshape(NCHUNKS, CHUNK)
    col2d = jnp.concatenate([edge_index[1], pad]).reshape(NCHUNKS, CHUNK)
    xpad = jnp.pad(x, ((0, NPAD - N), (0, 0)))
    zeros8 = jnp.zeros((ROWS_PER_TILE, DEGW), jnp.float32)
    ones8 = jnp.ones((CHUNK, DEGW), jnp.float32)
    zerosd = jnp.zeros((ROWS_PER_TILE, D), jnp.float32)
    degp = _deg_call(col2d, zeros8, ones8)
    xs = _prescale(degp, xpad)
    aggp = _agg_call(row2d, col2d, xs, zerosd)
    return _final(degp, aggp, W_w, W_b.reshape(1, D))[:N]


# speedup vs baseline: 1.0043x; 1.0043x over previous
import jax
import jax.numpy as jnp
from jax import lax
from jax.experimental import pallas as pl
from jax.experimental.pallas import tpu as pltpu
from jax.experimental.pallas import tpu_sc as plsc

N = 10000
NPAD = 10240
E = 320000
D = 128

NC = 2
NS = 16
NW = NC * NS

CHUNK = 64
P = 160
NCHUNKS = NW * P
EPAD = NCHUNKS * CHUNK
NB = 1
GROUP = NB * CHUNK
NG = P // NB
ROWS_PER_TILE = NPAD // NS
DEGW = 16
DEGWIN = 16

_mesh = plsc.VectorSubcoreMesh(core_axis_name="c", subcore_axis_name="s")

_sc_params = pltpu.CompilerParams(use_tc_tiling_on_sc=False)


def _deg_body(col2d_hbm, zeros8_hbm, ones8_hbm, degp_hbm,
              deg_sh, cidx_v, ones_v, dsem):
    c = lax.axis_index("c")
    s = lax.axis_index("s")
    wid = s * NC + c
    pltpu.sync_copy(zeros8_hbm, deg_sh.at[pl.ds(s * ROWS_PER_TILE, ROWS_PER_TILE)])
    pltpu.sync_copy(ones8_hbm, ones_v)
    pltpu.sync_copy(col2d_hbm.at[pl.ds(wid * P, P)], cidx_v)
    plsc.subcore_barrier()

    def body(j, _):
        @pl.when(j < P)
        def _():
            pltpu.async_copy(ones_v, deg_sh.at[cidx_v.at[j]], dsem, add=True)

        @pl.when(j >= DEGWIN)
        def _():
            pltpu.make_async_copy(ones8_hbm, ones_v, dsem).wait()

        return None

    lax.fori_loop(0, P + DEGWIN, body, None)
    plsc.subcore_barrier()
    pltpu.sync_copy(
        deg_sh.at[pl.ds(s * ROWS_PER_TILE, ROWS_PER_TILE)],
        degp_hbm.at[c, pl.ds(s * ROWS_PER_TILE, ROWS_PER_TILE)],
    )


_deg_call = pl.kernel(
    _deg_body,
    out_type=jax.ShapeDtypeStruct((NC, NPAD, DEGW), jnp.float32),
    mesh=_mesh,
    scratch_types=[
        pltpu.VMEM_SHARED((NPAD, DEGW), jnp.float32),
        pltpu.VMEM((P, CHUNK), jnp.int32),
        pltpu.VMEM((CHUNK, DEGW), jnp.float32),
        pltpu.SemaphoreType.DMA,
    ],
    compiler_params=_sc_params,
)


def _agg_body(row2d_hbm, col2d_hbm, xs_hbm, zerosd_hbm, aggp_hbm,
              acc_sh, ridx_v, cidx_v, rows_v, gsem0, gsem1, ssem0, ssem1):
    c = lax.axis_index("c")
    s = lax.axis_index("s")
    wid = s * NC + c
    pltpu.sync_copy(row2d_hbm.at[pl.ds(wid * P, P)], ridx_v)
    pltpu.sync_copy(col2d_hbm.at[pl.ds(wid * P, P)], cidx_v)
    pltpu.sync_copy(zerosd_hbm, acc_sh.at[pl.ds(s * ROWS_PER_TILE, ROWS_PER_TILE)])
    plsc.subcore_barrier()

    def gather_group(g, slot, gsem):
        for k in range(NB):
            pltpu.async_copy(
                xs_hbm.at[ridx_v.at[g * NB + k]],
                rows_v.at[slot, pl.ds(k * CHUNK, CHUNK)],
                gsem,
            )

    def scatter_group(g, slot, ssem):
        for k in range(NB):
            pltpu.async_copy(
                rows_v.at[slot, pl.ds(k * CHUNK, CHUNK)],
                acc_sh.at[cidx_v.at[g * NB + k]],
                ssem,
                add=True,
            )

    def drain(slot, sem):
        pltpu.make_async_copy(xs_hbm.at[pl.ds(0, GROUP)], rows_v.at[slot], sem).wait()

    gather_group(0, 0, gsem0)

    def real_body(t, _):
        g0 = 2 * t
        g1 = g0 + 1

        @pl.when(t >= 1)
        def _():
            drain(1, ssem1)

        gather_group(g1, 1, gsem1)
        drain(0, gsem0)
        scatter_group(g0, 0, ssem0)

        drain(0, ssem0)

        @pl.when(t < NG // 2 - 1)
        def _():
            gather_group(g1 + 1, 0, gsem0)

        drain(1, gsem1)
        scatter_group(g1, 1, ssem1)
        return None

    lax.fori_loop(0, NG // 2, real_body, None)
    drain(1, ssem1)
    plsc.subcore_barrier()
    pltpu.sync_copy(
        acc_sh.at[pl.ds(s * ROWS_PER_TILE, ROWS_PER_TILE)],
        aggp_hbm.at[c, pl.ds(s * ROWS_PER_TILE, ROWS_PER_TILE)],
    )


_agg_call = pl.kernel(
    _agg_body,
    out_type=jax.ShapeDtypeStruct((NC, NPAD, D), jnp.float32),
    mesh=_mesh,
    scratch_types=[
        pltpu.VMEM_SHARED((NPAD, D), jnp.float32),
        pltpu.VMEM((P, CHUNK), jnp.int32),
        pltpu.VMEM((P, CHUNK), jnp.int32),
        pltpu.VMEM((2, GROUP, D), jnp.float32),
        pltpu.SemaphoreType.DMA,
        pltpu.SemaphoreType.DMA,
        pltpu.SemaphoreType.DMA,
        pltpu.SemaphoreType.DMA,
    ],
    compiler_params=_sc_params,
)

_BN = 1000


def _prescale_body(degp_ref, x_ref, xs_ref):
    deg = degp_ref[0, :, 0:1] + degp_ref[1, :, 0:1]
    dis = jnp.where(deg > 0.0, lax.rsqrt(deg), 0.0)
    xs_ref[...] = x_ref[...] * dis


def _prescale(degp, x):
    return pl.pallas_call(
        _prescale_body,
        grid=(N // _BN,),
        in_specs=[
            pl.BlockSpec((NC, _BN, DEGW), lambda i: (0, i, 0)),
            pl.BlockSpec((_BN, D), lambda i: (i, 0)),
        ],
        out_specs=pl.BlockSpec((_BN, D), lambda i: (i, 0)),
        out_shape=jax.ShapeDtypeStruct((N, D), jnp.float32),
    )(degp, x)


def _final_body(degp_ref, aggp_ref, w_ref, b_ref, o_ref):
    deg = degp_ref[0, :, 0:1] + degp_ref[1, :, 0:1]
    dis = jnp.where(deg > 0.0, lax.rsqrt(deg), 0.0)
    acc = (aggp_ref[0] + aggp_ref[1]) * dis
    o_ref[...] = lax.dot_general(
        acc, w_ref[...], (((1,), (1,)), ((), ())),
        preferred_element_type=jnp.float32,
    ) + b_ref[...]


def _final(degp, aggp, W_w, W_b2):
    return pl.pallas_call(
        _final_body,
        grid=(N // _BN,),
        in_specs=[
            pl.BlockSpec((NC, _BN, DEGW), lambda i: (0, i, 0)),
            pl.BlockSpec((NC, _BN, D), lambda i: (0, i, 0)),
            pl.BlockSpec((D, D), lambda i: (0, 0)),
            pl.BlockSpec((1, D), lambda i: (0, 0)),
        ],
        out_specs=pl.BlockSpec((_BN, D), lambda i: (i, 0)),
        out_shape=jax.ShapeDtypeStruct((N, D), jnp.float32),
    )(degp, aggp, W_w, W_b2)


@jax.jit
def kernel(x, edge_index, x0, W_w, W_b):
    del x0
    padr = jnp.zeros((EPAD - E,), jnp.int32)
    padc = N + jnp.arange(EPAD - E, dtype=jnp.int32) % (NPAD - N)
    row2d = jnp.concatenate([edge_index[0], padr]).reshape(NCHUNKS, CHUNK)
    col2d = jnp.concatenate([edge_index[1], padc]).reshape(NCHUNKS, CHUNK)
    zeros8 = jnp.zeros((ROWS_PER_TILE, DEGW), jnp.float32)
    ones8 = jnp.ones((CHUNK, DEGW), jnp.float32)
    zerosd = jnp.zeros((ROWS_PER_TILE, D), jnp.float32)
    degp = _deg_call(col2d, zeros8, ones8)
    xs = _prescale(degp, x)
    aggp = _agg_call(row2d, col2d, xs, zerosd)
    return _final(degp, aggp, W_w, W_b.reshape(1, D))


# speedup vs baseline: 2.4637x; 2.4532x over previous
import jax
import jax.numpy as jnp
from jax import lax
from jax.experimental import pallas as pl
from jax.experimental.pallas import tpu as pltpu
from jax.experimental.pallas import tpu_sc as plsc

N = 10000
NPAD = 10240
E = 320000
D = 128

NC = 2
NS = 16
NW = NC * NS

CHUNK = 64
P = 160
NCHUNKS = NW * P
EPAD = NCHUNKS * CHUNK
NB = 1
GROUP = NB * CHUNK
NG = P // NB
ROWS_PER_TILE = NPAD // NS
DEGW = 16
DEGWIN = 16

_mesh = plsc.VectorSubcoreMesh(core_axis_name="c", subcore_axis_name="s")

_sc_params = pltpu.CompilerParams(use_tc_tiling_on_sc=False)


def _deg_body(col2d_hbm, zeros8_hbm, ones8_hbm, degp_hbm,
              deg_sh, cidx_v, ones_v, dsem):
    c = lax.axis_index("c")
    s = lax.axis_index("s")
    wid = s * NC + c
    pltpu.sync_copy(zeros8_hbm, deg_sh.at[pl.ds(s * ROWS_PER_TILE, ROWS_PER_TILE)])
    pltpu.sync_copy(ones8_hbm, ones_v)
    pltpu.sync_copy(col2d_hbm.at[pl.ds(wid * P, P)], cidx_v)
    plsc.subcore_barrier()

    def body(j, _):
        @pl.when(j < P)
        def _():
            pltpu.async_copy(ones_v, deg_sh.at[cidx_v.at[j]], dsem, add=True)

        @pl.when(j >= DEGWIN)
        def _():
            pltpu.make_async_copy(ones8_hbm, ones_v, dsem).wait()

        return None

    lax.fori_loop(0, P + DEGWIN, body, None)
    plsc.subcore_barrier()
    pltpu.sync_copy(
        deg_sh.at[pl.ds(s * ROWS_PER_TILE, ROWS_PER_TILE)],
        degp_hbm.at[c, pl.ds(s * ROWS_PER_TILE, ROWS_PER_TILE)],
    )


_deg_call = pl.kernel(
    _deg_body,
    out_type=jax.ShapeDtypeStruct((NC, NPAD, DEGW), jnp.float32),
    mesh=_mesh,
    scratch_types=[
        pltpu.VMEM_SHARED((NPAD, DEGW), jnp.float32),
        pltpu.VMEM((P, CHUNK), jnp.int32),
        pltpu.VMEM((CHUNK, DEGW), jnp.float32),
        pltpu.SemaphoreType.DMA,
    ],
    compiler_params=_sc_params,
)


def _agg_body(row2d_hbm, col2d_hbm, xs_hbm, zerosd_hbm, aggp_hbm,
              acc_sh, ridx_v, cidx_v, rows_v, gsem0, gsem1, ssem0, ssem1):
    c = lax.axis_index("c")
    s = lax.axis_index("s")
    wid = s * NC + c
    pltpu.sync_copy(row2d_hbm.at[pl.ds(wid * P, P)], ridx_v)
    pltpu.sync_copy(col2d_hbm.at[pl.ds(wid * P, P)], cidx_v)
    pltpu.sync_copy(zerosd_hbm, acc_sh.at[pl.ds(s * ROWS_PER_TILE, ROWS_PER_TILE)])
    plsc.subcore_barrier()

    def gather_group(g, slot, gsem):
        for k in range(NB):
            pltpu.async_copy(
                xs_hbm.at[ridx_v.at[g * NB + k]],
                rows_v.at[slot, pl.ds(k * CHUNK, CHUNK)],
                gsem,
            )

    def scatter_group(g, slot, ssem):
        for k in range(NB):
            pltpu.async_copy(
                rows_v.at[slot, pl.ds(k * CHUNK, CHUNK)],
                acc_sh.at[cidx_v.at[g * NB + k]],
                ssem,
                add=True,
            )

    def drain(slot, sem):
        pltpu.make_async_copy(xs_hbm.at[pl.ds(0, GROUP)], rows_v.at[slot], sem).wait()

    gather_group(0, 0, gsem0)

    def real_body(t, _):
        g0 = 2 * t
        g1 = g0 + 1

        @pl.when(t >= 1)
        def _():
            drain(1, ssem1)

        gather_group(g1, 1, gsem1)
        drain(0, gsem0)
        scatter_group(g0, 0, ssem0)

        drain(0, ssem0)

        @pl.when(t < NG // 2 - 1)
        def _():
            gather_group(g1 + 1, 0, gsem0)

        drain(1, gsem1)
        scatter_group(g1, 1, ssem1)
        return None

    lax.fori_loop(0, NG // 2, real_body, None)
    drain(1, ssem1)
    plsc.subcore_barrier()
    pltpu.sync_copy(
        acc_sh.at[pl.ds(s * ROWS_PER_TILE, ROWS_PER_TILE)],
        aggp_hbm.at[c, pl.ds(s * ROWS_PER_TILE, ROWS_PER_TILE)],
    )


_agg_call = pl.kernel(
    _agg_body,
    out_type=jax.ShapeDtypeStruct((NC, NPAD, D), jnp.float32),
    mesh=_mesh,
    scratch_types=[
        pltpu.VMEM_SHARED((NPAD, D), jnp.float32),
        pltpu.VMEM((P, CHUNK), jnp.int32),
        pltpu.VMEM((P, CHUNK), jnp.int32),
        pltpu.VMEM((2, GROUP, D), jnp.float32),
        pltpu.SemaphoreType.DMA,
        pltpu.SemaphoreType.DMA,
        pltpu.SemaphoreType.DMA,
        pltpu.SemaphoreType.DMA,
    ],
    compiler_params=_sc_params,
)

_BN = 1000


def _prescale_body(degp_ref, x_ref, xs_ref):
    deg = degp_ref[0, :, 0:1] + degp_ref[1, :, 0:1]
    dis = jnp.where(deg > 0.0, lax.rsqrt(deg), 0.0)
    xs_ref[...] = x_ref[...] * dis


def _prescale(degp, x):
    return pl.pallas_call(
        _prescale_body,
        grid=(N // _BN,),
        in_specs=[
            pl.BlockSpec((NC, _BN, DEGW), lambda i: (0, i, 0)),
            pl.BlockSpec((_BN, D), lambda i: (i, 0)),
        ],
        out_specs=pl.BlockSpec((_BN, D), lambda i: (i, 0)),
        out_shape=jax.ShapeDtypeStruct((N, D), jnp.float32),
    )(degp, x)


def _final_body(degp_ref, aggp_ref, w_ref, b_ref, o_ref):
    deg = degp_ref[0, :, 0:1] + degp_ref[1, :, 0:1]
    dis = jnp.where(deg > 0.0, lax.rsqrt(deg), 0.0)
    acc = (aggp_ref[0] + aggp_ref[1]) * dis
    o_ref[...] = lax.dot_general(
        acc, w_ref[...], (((1,), (1,)), ((), ())),
        preferred_element_type=jnp.float32,
    ) + b_ref[...]


def _final(degp, aggp, W_w, W_b2):
    return pl.pallas_call(
        _final_body,
        grid=(N // _BN,),
        in_specs=[
            pl.BlockSpec((NC, _BN, DEGW), lambda i: (0, i, 0)),
            pl.BlockSpec((NC, _BN, D), lambda i: (0, i, 0)),
            pl.BlockSpec((D, D), lambda i: (0, 0)),
            pl.BlockSpec((1, D), lambda i: (0, 0)),
        ],
        out_specs=pl.BlockSpec((_BN, D), lambda i: (i, 0)),
        out_shape=jax.ShapeDtypeStruct((N, D), jnp.float32),
    )(degp, aggp, W_w, W_b2)


@jax.jit
def kernel(x, edge_index, x0, W_w, W_b):
    del x0
    padr = jnp.arange(EPAD - E, dtype=jnp.int32) % N
    padc = N + jnp.arange(EPAD - E, dtype=jnp.int32) % (NPAD - N)
    row2d = jnp.concatenate([edge_index[0], padr]).reshape(NCHUNKS, CHUNK)
    col2d = jnp.concatenate([edge_index[1], padc]).reshape(NCHUNKS, CHUNK)
    zeros8 = jnp.zeros((ROWS_PER_TILE, DEGW), jnp.float32)
    ones8 = jnp.ones((CHUNK, DEGW), jnp.float32)
    zerosd = jnp.zeros((ROWS_PER_TILE, D), jnp.float32)
    degp = _deg_call(col2d, zeros8, ones8)
    xs = _prescale(degp, x)
    aggp = _agg_call(row2d, col2d, xs, zerosd)
    return _final(degp, aggp, W_w, W_b.reshape(1, D))


# speedup vs baseline: 5.5225x; 2.2415x over previous
import jax
import jax.numpy as jnp
from jax import lax
from jax.experimental import pallas as pl
from jax.experimental.pallas import tpu as pltpu
from jax.experimental.pallas import tpu_sc as plsc

N = 10000
NPAD = 10240
E = 320000
D = 128

NC = 2
NS = 16
NW = NC * NS

CHUNK = 64
P = 160
NCHUNKS = NW * P
EPAD = NCHUNKS * CHUNK
NB = 1
GROUP = NB * CHUNK
NG = P // NB
ROWS_PER_TILE = NPAD // NS
DEGW = 16
DEGWIN = 16

_mesh = plsc.VectorSubcoreMesh(core_axis_name="c", subcore_axis_name="s")

_sc_params = pltpu.CompilerParams(use_tc_tiling_on_sc=False)


def _deg_body(ei3_hbm, zeros8_hbm, ones8_hbm, degp_hbm,
              deg_sh, cidx_v, ones_v, dsem):
    c = lax.axis_index("c")
    s = lax.axis_index("s")
    wid = s * NC + c
    pltpu.sync_copy(zeros8_hbm, deg_sh.at[pl.ds(s * ROWS_PER_TILE, ROWS_PER_TILE)])
    pltpu.sync_copy(ones8_hbm, ones_v)
    pltpu.sync_copy(ei3_hbm.at[1, pl.ds(wid * P, P)], cidx_v)
    plsc.subcore_barrier()

    def body(j, _):
        @pl.when(j < P)
        def _():
            pltpu.async_copy(ones_v, deg_sh.at[cidx_v.at[j]], dsem, add=True)

        @pl.when(j >= DEGWIN)
        def _():
            pltpu.make_async_copy(ones8_hbm, ones_v, dsem).wait()

        return None

    lax.fori_loop(0, P + DEGWIN, body, None)
    plsc.subcore_barrier()
    pltpu.sync_copy(
        deg_sh.at[pl.ds(s * ROWS_PER_TILE, ROWS_PER_TILE), pl.ds(0, 8)],
        degp_hbm.at[c, pl.ds(s * ROWS_PER_TILE, ROWS_PER_TILE)],
    )


_deg_call = pl.kernel(
    _deg_body,
    out_type=jax.ShapeDtypeStruct((NC, NPAD, 8), jnp.float32),
    mesh=_mesh,
    scratch_types=[
        pltpu.VMEM_SHARED((NPAD, DEGW), jnp.float32),
        pltpu.VMEM((P, CHUNK), jnp.int32),
        pltpu.VMEM((CHUNK, DEGW), jnp.float32),
        pltpu.SemaphoreType.DMA,
    ],
    compiler_params=_sc_params,
)


def _agg_body(ei3_hbm, xs_hbm, zerosd_hbm, aggp_hbm,
              acc_sh, ridx_v, cidx_v, rows_v, gsem0, gsem1, ssem0, ssem1):
    c = lax.axis_index("c")
    s = lax.axis_index("s")
    wid = s * NC + c
    pltpu.sync_copy(ei3_hbm.at[0, pl.ds(wid * P, P)], ridx_v)
    pltpu.sync_copy(ei3_hbm.at[1, pl.ds(wid * P, P)], cidx_v)
    pltpu.sync_copy(zerosd_hbm, acc_sh.at[pl.ds(s * ROWS_PER_TILE, ROWS_PER_TILE)])
    plsc.subcore_barrier()

    def gather_group(g, slot, gsem):
        for k in range(NB):
            pltpu.async_copy(
                xs_hbm.at[ridx_v.at[g * NB + k]],
                rows_v.at[slot, pl.ds(k * CHUNK, CHUNK)],
                gsem,
            )

    def scatter_group(g, slot, ssem):
        for k in range(NB):
            pltpu.async_copy(
                rows_v.at[slot, pl.ds(k * CHUNK, CHUNK)],
                acc_sh.at[cidx_v.at[g * NB + k]],
                ssem,
                add=True,
            )

    def drain(slot, sem):
        pltpu.make_async_copy(xs_hbm.at[pl.ds(0, GROUP)], rows_v.at[slot], sem).wait()

    gather_group(0, 0, gsem0)

    def real_body(t, _):
        g0 = 2 * t
        g1 = g0 + 1

        @pl.when(t >= 1)
        def _():
            drain(1, ssem1)

        gather_group(g1, 1, gsem1)
        drain(0, gsem0)
        scatter_group(g0, 0, ssem0)

        drain(0, ssem0)

        @pl.when(t < NG // 2 - 1)
        def _():
            gather_group(g1 + 1, 0, gsem0)

        drain(1, gsem1)
        scatter_group(g1, 1, ssem1)
        return None

    lax.fori_loop(0, NG // 2, real_body, None)
    drain(1, ssem1)
    plsc.subcore_barrier()
    pltpu.sync_copy(
        acc_sh.at[pl.ds(s * ROWS_PER_TILE, ROWS_PER_TILE)],
        aggp_hbm.at[c, pl.ds(s * ROWS_PER_TILE, ROWS_PER_TILE)],
    )


_agg_call = pl.kernel(
    _agg_body,
    out_type=jax.ShapeDtypeStruct((NC, NPAD, D), jnp.float32),
    mesh=_mesh,
    scratch_types=[
        pltpu.VMEM_SHARED((NPAD, D), jnp.float32),
        pltpu.VMEM((P, CHUNK), jnp.int32),
        pltpu.VMEM((P, CHUNK), jnp.int32),
        pltpu.VMEM((2, GROUP, D), jnp.float32),
        pltpu.SemaphoreType.DMA,
        pltpu.SemaphoreType.DMA,
        pltpu.SemaphoreType.DMA,
        pltpu.SemaphoreType.DMA,
    ],
    compiler_params=_sc_params,
)

_BN = 1000


def _prescale_body(degp_ref, x_ref, xs_ref):
    deg = degp_ref[0, :, 0:1] + degp_ref[1, :, 0:1]
    dis = jnp.where(deg > 0.0, lax.rsqrt(deg), 0.0)
    xs_ref[...] = x_ref[...] * dis


def _prescale(degp, x):
    return pl.pallas_call(
        _prescale_body,
        grid=(N // _BN,),
        in_specs=[
            pl.BlockSpec((NC, _BN, 8), lambda i: (0, i, 0)),
            pl.BlockSpec((_BN, D), lambda i: (i, 0)),
        ],
        out_specs=pl.BlockSpec((_BN, D), lambda i: (i, 0)),
        out_shape=jax.ShapeDtypeStruct((N, D), jnp.float32),
    )(degp, x)


def _final_body(degp_ref, aggp_ref, w_ref, b_ref, o_ref):
    deg = degp_ref[0, :, 0:1] + degp_ref[1, :, 0:1]
    dis = jnp.where(deg > 0.0, lax.rsqrt(deg), 0.0)
    acc = (aggp_ref[0] + aggp_ref[1]) * dis
    o_ref[...] = lax.dot_general(
        acc, w_ref[...], (((1,), (1,)), ((), ())),
        preferred_element_type=jnp.float32,
    ) + b_ref[...]


def _final(degp, aggp, W_w, W_b2):
    return pl.pallas_call(
        _final_body,
        grid=(N // _BN,),
        in_specs=[
            pl.BlockSpec((NC, _BN, 8), lambda i: (0, i, 0)),
            pl.BlockSpec((NC, _BN, D), lambda i: (0, i, 0)),
            pl.BlockSpec((D, D), lambda i: (0, 0)),
            pl.BlockSpec((1, D), lambda i: (0, 0)),
        ],
        out_specs=pl.BlockSpec((_BN, D), lambda i: (i, 0)),
        out_shape=jax.ShapeDtypeStruct((N, D), jnp.float32),
    )(degp, aggp, W_w, W_b2)


@jax.jit
def kernel(x, edge_index, x0, W_w, W_b):
    del x0
    padr = jnp.arange(EPAD - E, dtype=jnp.int32) % N
    padc = N + jnp.arange(EPAD - E, dtype=jnp.int32) % (NPAD - N)
    ei3 = jnp.concatenate(
        [edge_index, jnp.stack([padr, padc])], axis=1
    ).reshape(2, NCHUNKS, CHUNK)
    zeros8 = jnp.zeros((ROWS_PER_TILE, DEGW), jnp.float32)
    ones8 = jnp.ones((CHUNK, DEGW), jnp.float32)
    zerosd = jnp.zeros((ROWS_PER_TILE, D), jnp.float32)
    degp = _deg_call(ei3, zeros8, ones8)
    xs = _prescale(degp, x)
    aggp = _agg_call(ei3, xs, zerosd)
    return _final(degp, aggp, W_w, W_b.reshape(1, D))
